# baseline (speedup 1.0000x reference)
import functools

import jax
import jax.numpy as jnp
from jax import lax
from jax.experimental import pallas as pl
from jax.experimental.pallas import tpu as pltpu
from jax.experimental.pallas import tpu_sc as plsc

E = 50000
V = 16667
EPAD = 50176
BE = 512
NBLK = EPAD // BE
NW = 32
RPW = 4 * EPAD // NW
CHUNK = 112
NCHUNK = RPW // CHUNK
VW = 528
VPAD = NW * VW
WIN = 272
EPS = 1e-5

_SC_MESH = dict(core_axis_name="c", subcore_axis_name="s",
                num_cores=2, num_subcores=16)


def _pad_c(c):
    return max(16, c)


def _grp_of(cp):
    return 4 if cp > 64 else 7


def _gather_body(ntab, cp, *refs):
    tabs = refs[:ntab]
    idxs = refs[ntab]
    outs = refs[ntab + 1:2 * ntab + 1]
    idx_v, buf0, buf1, gs0, gs1, ss0, ss1 = refs[2 * ntab + 1:]
    grp = _grp_of(cp)
    ngrp = NCHUNK // grp
    rows_g = grp * CHUNK
    w = lax.axis_index("s") * 2 + lax.axis_index("c")
    pltpu.sync_copy(idxs.at[w], idx_v)
    last = ngrp - 1

    def fire(g, t, buf, sem):
        g = jnp.minimum(g, last)
        for b in range(grp):
            pltpu.async_copy(tabs[t].at[idx_v.at[g * grp + b]],
                             buf.at[pl.ds(b * CHUNK, CHUNK)], sem)

    def drain(g, t, buf, sem):
        g = jnp.minimum(g, last)
        for b in range(grp):
            pltpu.make_async_copy(tabs[t].at[idx_v.at[g * grp + b]],
                                  buf.at[pl.ds(b * CHUNK, CHUNK)], sem).wait()

    def astore(g, t, buf, sem):
        return pltpu.async_copy(
            buf, outs[t].at[pl.ds(w * RPW + g * rows_g, rows_g)], sem)

    def wstore(g, t, buf, sem):
        pltpu.make_async_copy(
            buf, outs[t].at[pl.ds(w * RPW + g * rows_g, rows_g)], sem).wait()

    if ntab == 1:
        fire(jnp.int32(0), 0, buf0, gs0)
        fire(jnp.int32(1), 0, buf1, gs1)

        def body(gi, carry):
            a = 2 * gi
            drain(a, 0, buf0, gs0)
            astore(a, 0, buf0, ss0)
            drain(a + 1, 0, buf1, gs1)
            astore(a + 1, 0, buf1, ss1)
            wstore(a, 0, buf0, ss0)
            fire(a + 2, 0, buf0, gs0)
            wstore(a + 1, 0, buf1, ss1)
            fire(a + 3, 0, buf1, gs1)
            return carry

        lax.fori_loop(0, ngrp // 2, body, 0)
        drain(jnp.int32(last), 0, buf0, gs0)
        drain(jnp.int32(last), 0, buf1, gs1)
    else:
        fire(jnp.int32(0), 0, buf0, gs0)
        fire(jnp.int32(0), 1, buf1, gs1)

        def body(gi, carry):
            drain(gi, 0, buf0, gs0)
            astore(gi, 0, buf0, ss0)
            drain(gi, 1, buf1, gs1)
            astore(gi, 1, buf1, ss1)
            wstore(gi, 0, buf0, ss0)
            fire(gi + 1, 0, buf0, gs0)
            wstore(gi, 1, buf1, ss1)
            fire(gi + 1, 1, buf1, gs1)
            return carry

        lax.fori_loop(0, ngrp, body, 0)
        drain(jnp.int32(last), 0, buf0, gs0)
        drain(jnp.int32(last), 1, buf1, gs1)


def _sc_gather(tables, idx3):
    ntab = len(tables)
    cps = [t.shape[1] for t in tables]
    assert len(set(cps)) == 1
    cp = cps[0]
    grp = _grp_of(cp)
    k = pl.kernel(
        functools.partial(_gather_body, ntab, cp),
        out_type=[jax.ShapeDtypeStruct((4 * EPAD, cp), jnp.float32)
                  for _ in cps],
        mesh=plsc.VectorSubcoreMesh(**_SC_MESH),
        scratch_types=[
            pltpu.VMEM((NCHUNK, CHUNK), jnp.int32),
            pltpu.VMEM((grp * CHUNK, cp), jnp.float32),
            pltpu.VMEM((grp * CHUNK, cp), jnp.float32),
            pltpu.SemaphoreType.DMA,
            pltpu.SemaphoreType.DMA,
            pltpu.SemaphoreType.DMA,
            pltpu.SemaphoreType.DMA,
        ],
        compiler_params=pltpu.CompilerParams(use_tc_tiling_on_sc=False),
    )
    outs = k(*tables, idx3)
    outs = list(outs) if isinstance(outs, (list, tuple)) else [outs]
    return [o.reshape(4, EPAD, cp) for o, cp in zip(outs, cps)]


def _T(g, st_ref, relu):
    y = (g - st_ref[0:1, :]) * st_ref[1:2, :]
    return jnp.maximum(y, 0.0) if relu else y


def _conv_body(cpi, mode, refs):
    if mode in ('norm2', 'norm2_inorm'):
        (in1, ngh1, st1, in2, ngh2, st2, *rest) = refs
    else:
        (in1, ngh1, st1, *rest) = refs
        in2 = ngh2 = st2 = None
    if mode == 'norm2_inorm':
        sta, w2_ref, b_ref, pre_ref, stout_ref, acc = rest
    else:
        sta = None
        w2_ref, b_ref, pre_ref, stout_ref, acc = rest

    def feat(k):
        if k == 0:
            g1 = in1[...]
            g2 = in2[...] if in2 is not None else None
        else:
            g1 = ngh1[k - 1]
            g2 = ngh2[k - 1] if ngh2 is not None else None
        if mode == 'raw':
            return g1
        if mode == 'norm':
            return _T(g1, st1, True)
        a = _T(g1, st1, True) + _T(g2, st2, True)
        if mode == 'norm2_inorm':
            a = (a - sta[0:1, :]) * sta[1:2, :]
        return a

    f0, f1, f2, f3, f4 = [feat(k) for k in range(5)]
    w2 = w2_ref[...]

    def dot(a, wk):
        return lax.dot_general(a, wk, (((1,), (0,)), ((), ())),
                               preferred_element_type=jnp.float32)

    pre = (dot(f0, w2[0:cpi])
           + dot(f1 + f3, w2[cpi:2 * cpi])
           + dot(f2 + f4, w2[2 * cpi:3 * cpi])
           + dot(jnp.abs(f1 - f3), w2[3 * cpi:4 * cpi])
           + dot(jnp.abs(f2 - f4), w2[4 * cpi:5 * cpi]))
    pre = pre + b_ref[0:1, :]
    pre_ref[...] = pre
    i = pl.program_id(0)
    rows = i * BE + lax.broadcasted_iota(jnp.int32, (BE, 1), 0)
    pm = jnp.where(rows < E, pre, 0.0)

    @pl.when(i == 0)
    def _():
        acc[...] = jnp.zeros_like(acc)

    acc[0:1, :] += jnp.sum(pm, axis=0).reshape(1, -1)
    acc[1:2, :] += jnp.sum(pm * pm, axis=0).reshape(1, -1)
    m = acc[0:1, :] * (1.0 / E)
    v = acc[1:2, :] * (1.0 / E) - m * m
    r = lax.rsqrt(v + EPS)
    cpo = m.shape[1]
    stout_ref[...] = jnp.concatenate(
        [m, r, jnp.zeros((6, cpo), jnp.float32)], axis=0)


def _tc_conv(mode, t1, t2, sta, w2, b2):
    in1, ngh1, st1 = t1
    cpi = in1.shape[1]
    cpo = w2.shape[1]
    in_specs = [
        pl.BlockSpec((BE, cpi), lambda i: (i, 0)),
        pl.BlockSpec((4, BE, cpi), lambda i: (0, i, 0)),
        pl.BlockSpec((8, cpi), lambda i: (0, 0)),
    ]
    args = [in1, ngh1, st1]
    if t2 is not None:
        in_specs = in_specs + [
            pl.BlockSpec((BE, cpi), lambda i: (i, 0)),
            pl.BlockSpec((4, BE, cpi), lambda i: (0, i, 0)),
            pl.BlockSpec((8, cpi), lambda i: (0, 0)),
        ]
        args += list(t2)
    if sta is not None:
        in_specs.append(pl.BlockSpec((8, cpi), lambda i: (0, 0)))
        args.append(sta)
    in_specs += [
        pl.BlockSpec((5 * cpi, cpo), lambda i: (0, 0)),
        pl.BlockSpec((8, cpo), lambda i: (0, 0)),
    ]
    args += [w2, b2]
    return pl.pallas_call(
        lambda *refs: _conv_body(cpi, mode, refs),
        grid=(NBLK,),
        in_specs=in_specs,
        out_specs=[
            pl.BlockSpec((BE, cpo), lambda i: (i, 0)),
            pl.BlockSpec((8, cpo), lambda i: (0, 0)),
        ],
        out_shape=[
            jax.ShapeDtypeStruct((EPAD, cpo), jnp.float32),
            jax.ShapeDtypeStruct((8, cpo), jnp.float32),
        ],
        scratch_shapes=[pltpu.VMEM((8, cpo), jnp.float32)],
    )(*args)


def _stats_body(p1, st1, p2, st2, stout_ref, acc):
    y = _T(p1[...], st1, True) + _T(p2[...], st2, True)
    i = pl.program_id(0)
    rows = i * BE + lax.broadcasted_iota(jnp.int32, (BE, 1), 0)
    ym = jnp.where(rows < E, y, 0.0)

    @pl.when(i == 0)
    def _():
        acc[...] = jnp.zeros_like(acc)

    acc[0:1, :] += jnp.sum(ym, axis=0).reshape(1, -1)
    acc[1:2, :] += jnp.sum(ym * ym, axis=0).reshape(1, -1)
    m = acc[0:1, :] * (1.0 / E)
    v = acc[1:2, :] * (1.0 / E) - m * m
    r = lax.rsqrt(v + EPS)
    cpo = m.shape[1]
    stout_ref[...] = jnp.concatenate(
        [m, r, jnp.zeros((6, cpo), jnp.float32)], axis=0)


def _tc_stats2(p1, st1, p2, st2):
    cp = p1.shape[1]
    return pl.pallas_call(
        _stats_body,
        grid=(NBLK,),
        in_specs=[
            pl.BlockSpec((BE, cp), lambda i: (i, 0)),
            pl.BlockSpec((8, cp), lambda i: (0, 0)),
            pl.BlockSpec((BE, cp), lambda i: (i, 0)),
            pl.BlockSpec((8, cp), lambda i: (0, 0)),
        ],
        out_specs=pl.BlockSpec((8, cp), lambda i: (0, 0)),
        out_shape=jax.ShapeDtypeStruct((8, cp), jnp.float32),
        scratch_shapes=[pltpu.VMEM((8, cp), jnp.float32)],
    )(p1, st1, p2, st2)


def _buildv_body(fe, nvsp, out, win_v, nvs_v, out_v, sem):
    w = lax.axis_index("s") * 2 + lax.axis_index("c")
    v0 = w * VW
    starts = []
    for k in range(6):
        st = ((v0 + k * V) // 2) & ~7
        starts.append(st)
        pltpu.async_copy(fe.at[pl.ds(st, WIN)], win_v.at[k], sem)
    pltpu.sync_copy(nvsp.at[pl.ds(v0, VW)], nvs_v)
    for k in range(6):
        pltpu.make_async_copy(fe.at[pl.ds(starts[k], WIN)], win_v.at[k],
                              sem).wait()

    def it(t, carry):
        vv = v0 + t * 16 + lax.iota(jnp.int32, 16)
        invn = nvs_v[pl.ds(t * 16, 16)]
        for c in range(3):
            acc = jnp.zeros((16,), jnp.float32)
            for k in range(6):
                j = vv + k * V
                valid = j < 2 * E
                e_loc = jnp.where(valid, (j >> 1) - starts[k], 0)
                ch = (j & 1) * 3 + c
                kv = jnp.full((16,), k, jnp.int32)
                vals = plsc.load_gather(win_v, [kv, e_loc, ch])
                acc = acc + jnp.where(valid, vals, 0.0)
            acc = acc / invn
            oidx = (t * 16 + lax.iota(jnp.int32, 16)) * 8 + c
            plsc.store_scatter(out_v, [oidx], acc)
        return carry

    lax.fori_loop(0, VW // 16, it, 0)
    pltpu.sync_copy(out_v, out.at[w])


def _sc_buildv(fe, nvsp):
    k = pl.kernel(
        _buildv_body,
        out_type=jax.ShapeDtypeStruct((NW, VW * 8), jnp.float32),
        mesh=plsc.VectorSubcoreMesh(**_SC_MESH),
        scratch_types=[
            pltpu.VMEM((6, WIN, 16), jnp.float32),
            pltpu.VMEM((VW,), jnp.float32),
            pltpu.VMEM((VW * 8,), jnp.float32),
            pltpu.SemaphoreType.DMA,
        ],
        compiler_params=pltpu.CompilerParams(use_tc_tiling_on_sc=False,
                                             needs_layout_passes=False),
    )
    return k(fe, nvsp)


def _prep_w(w, b):
    cout, cin, _ = w.shape
    cpi, cpo = _pad_c(cin), _pad_c(cout)
    w2 = jnp.zeros((5, cpi, cpo), jnp.float32)
    w2 = w2.at[:, :cin, :cout].set(jnp.transpose(w, (2, 1, 0)))
    b2 = jnp.zeros((8, cpo), jnp.float32).at[0, :cout].set(b)
    return w2.reshape(5 * cpi, cpo), b2


def kernel(x, t, gemm, vei, ve_in, nvsi, nvsin, nvs, params):
    del t, vei, ve_in, nvsi, nvsin
    idx3 = (jnp.zeros((4, EPAD), jnp.int32).at[:, :E].set(gemm[:, 1:].T)
            .reshape(NW, NCHUNK, CHUNK))
    act0 = jnp.zeros((EPAD, 16), jnp.float32).at[:E, :6].set(x[0].T)
    st_id = jnp.zeros((8, 16), jnp.float32).at[1, :].set(1.0)

    def conv(mode, t1, t2, sta, p):
        w2, b2 = _prep_w(p['w'], p['b'])
        tabs = [t1[0]] if t2 is None else [t1[0], t2[0]]
        nghs = _sc_gather(tabs, idx3)
        a1 = (t1[0], nghs[0], t1[1])
        a2 = None if t2 is None else (t2[0], nghs[1], t2[1])
        return _tc_conv(mode, a1, a2, sta, w2, b2)

    state = [(act0, st_id)]
    mode0 = 'raw'
    blocks = list(params['down']) + list(params['up']) + [params['final']]
    for p in blocks:
        if len(state) == 1:
            p1, s1 = conv(mode0, state[0], None, None, p['c1'])
        else:
            p1, s1 = conv('norm2', state[0], state[1], None, p['c1'])
        mode0 = 'norm'
        p2, s2 = p1, s1
        for bp in p['blocks']:
            p2, s2 = conv('norm', (p1, s1), None, None, bp['conv'])
        state = [(p1, s1), (p2, s2)]

    sta = _tc_stats2(state[0][0], state[0][1], state[1][0], state[1][1])
    fe, _ = conv('norm2_inorm', state[0], state[1], sta, params['last'])

    nvsp = jnp.ones((VPAD,), jnp.float32).at[:V].set(nvs)
    outb = _sc_buildv(fe, nvsp)
    return outb.reshape(VPAD, 8)[:V, :3][None].astype(jnp.float32)


# speedup vs baseline: 1.1501x; 1.1501x over previous
import functools

import jax
import jax.numpy as jnp
from jax import lax
from jax.experimental import pallas as pl
from jax.experimental.pallas import tpu as pltpu
from jax.experimental.pallas import tpu_sc as plsc

E = 50000
V = 16667
EPAD = 50176
BE = 512
NBLK = EPAD // BE
NW = 32
RPW = 4 * EPAD // NW
CHUNK = 112
NCHUNK = RPW // CHUNK
VW = 528
VPAD = NW * VW
WIN = 272
EPS = 1e-5

_SC_MESH = dict(core_axis_name="c", subcore_axis_name="s",
                num_cores=2, num_subcores=16)


def _pad_c(c):
    return max(16, c)


def _grp_of(cp):
    return 4 if cp > 64 else 7


def _gather_body(ntab, cp, *refs):
    tabs = refs[:ntab]
    idxs = refs[ntab]
    outs = refs[ntab + 1:2 * ntab + 1]
    idx_v, buf0, buf1, gs0, gs1, ss0, ss1 = refs[2 * ntab + 1:]
    grp = _grp_of(cp)
    ngrp = NCHUNK // grp
    rows_g = grp * CHUNK
    w = lax.axis_index("s") * 2 + lax.axis_index("c")
    pltpu.sync_copy(idxs.at[w], idx_v)
    last = ngrp - 1

    def fire(g, t, buf, sem):
        g = jnp.minimum(g, last)
        for b in range(grp):
            pltpu.async_copy(tabs[t].at[idx_v.at[g * grp + b]],
                             buf.at[pl.ds(b * CHUNK, CHUNK)], sem)

    def drain(g, t, buf, sem):
        g = jnp.minimum(g, last)
        for b in range(grp):
            pltpu.make_async_copy(tabs[t].at[idx_v.at[g * grp + b]],
                                  buf.at[pl.ds(b * CHUNK, CHUNK)], sem).wait()

    def astore(g, t, buf, sem):
        return pltpu.async_copy(
            buf, outs[t].at[pl.ds(w * RPW + g * rows_g, rows_g)], sem)

    def wstore(g, t, buf, sem):
        pltpu.make_async_copy(
            buf, outs[t].at[pl.ds(w * RPW + g * rows_g, rows_g)], sem).wait()

    if ntab == 1:
        fire(jnp.int32(0), 0, buf0, gs0)
        fire(jnp.int32(1), 0, buf1, gs1)

        def body(gi, carry):
            a = 2 * gi
            drain(a, 0, buf0, gs0)
            astore(a, 0, buf0, ss0)
            drain(a + 1, 0, buf1, gs1)
            astore(a + 1, 0, buf1, ss1)
            wstore(a, 0, buf0, ss0)
            fire(a + 2, 0, buf0, gs0)
            wstore(a + 1, 0, buf1, ss1)
            fire(a + 3, 0, buf1, gs1)
            return carry

        lax.fori_loop(0, ngrp // 2, body, 0)
        drain(jnp.int32(last), 0, buf0, gs0)
        drain(jnp.int32(last), 0, buf1, gs1)
    else:
        fire(jnp.int32(0), 0, buf0, gs0)
        fire(jnp.int32(0), 1, buf1, gs1)

        def body(gi, carry):
            drain(gi, 0, buf0, gs0)
            astore(gi, 0, buf0, ss0)
            drain(gi, 1, buf1, gs1)
            astore(gi, 1, buf1, ss1)
            wstore(gi, 0, buf0, ss0)
            fire(gi + 1, 0, buf0, gs0)
            wstore(gi, 1, buf1, ss1)
            fire(gi + 1, 1, buf1, gs1)
            return carry

        lax.fori_loop(0, ngrp, body, 0)
        drain(jnp.int32(last), 0, buf0, gs0)
        drain(jnp.int32(last), 1, buf1, gs1)


def _sc_gather(tables, idx3):
    ntab = len(tables)
    cps = [t.shape[1] for t in tables]
    assert len(set(cps)) == 1
    cp = cps[0]
    grp = _grp_of(cp)
    k = pl.kernel(
        functools.partial(_gather_body, ntab, cp),
        out_type=[jax.ShapeDtypeStruct((4 * EPAD, cp), jnp.float32)
                  for _ in cps],
        mesh=plsc.VectorSubcoreMesh(**_SC_MESH),
        scratch_types=[
            pltpu.VMEM((NCHUNK, CHUNK), jnp.int32),
            pltpu.VMEM((grp * CHUNK, cp), jnp.float32),
            pltpu.VMEM((grp * CHUNK, cp), jnp.float32),
            pltpu.SemaphoreType.DMA,
            pltpu.SemaphoreType.DMA,
            pltpu.SemaphoreType.DMA,
            pltpu.SemaphoreType.DMA,
        ],
        compiler_params=pltpu.CompilerParams(use_tc_tiling_on_sc=False),
    )
    outs = k(*tables, idx3)
    outs = list(outs) if isinstance(outs, (list, tuple)) else [outs]
    return [o.reshape(4, EPAD, cp) for o, cp in zip(outs, cps)]


def _T(g, st_ref, relu):
    y = (g - st_ref[0:1, :]) * st_ref[1:2, :]
    return jnp.maximum(y, 0.0) if relu else y


def _conv_body(cpi, mode, refs):
    if mode in ('norm2', 'norm2_inorm'):
        (in1, ngh1, st1, in2, ngh2, st2, *rest) = refs
    else:
        (in1, ngh1, st1, *rest) = refs
        in2 = ngh2 = st2 = None
    if mode == 'norm2_inorm':
        sta, w2_ref, b_ref, pre_ref, stout_ref, acc = rest
    else:
        sta = None
        w2_ref, b_ref, pre_ref, stout_ref, acc = rest

    def feat(k):
        if k == 0:
            g1 = in1[...]
            g2 = in2[...] if in2 is not None else None
        else:
            g1 = ngh1[k - 1]
            g2 = ngh2[k - 1] if ngh2 is not None else None
        if mode == 'raw':
            return g1
        if mode == 'norm':
            return _T(g1, st1, True)
        a = _T(g1, st1, True) + _T(g2, st2, True)
        if mode == 'norm2_inorm':
            a = (a - sta[0:1, :]) * sta[1:2, :]
        return a

    f0, f1, f2, f3, f4 = [feat(k) for k in range(5)]
    w2 = w2_ref[...]

    def dot(a, wk):
        return lax.dot_general(a, wk, (((1,), (0,)), ((), ())),
                               preferred_element_type=jnp.float32)

    pre = (dot(f0, w2[0:cpi])
           + dot(f1 + f3, w2[cpi:2 * cpi])
           + dot(f2 + f4, w2[2 * cpi:3 * cpi])
           + dot(jnp.abs(f1 - f3), w2[3 * cpi:4 * cpi])
           + dot(jnp.abs(f2 - f4), w2[4 * cpi:5 * cpi]))
    pre = pre + b_ref[0:1, :]
    pre_ref[...] = pre
    i = pl.program_id(0)
    rows = i * BE + lax.broadcasted_iota(jnp.int32, (BE, 1), 0)
    pm = jnp.where(rows < E, pre, 0.0)

    @pl.when(i == 0)
    def _():
        acc[...] = jnp.zeros_like(acc)

    acc[0:1, :] += jnp.sum(pm, axis=0).reshape(1, -1)
    acc[1:2, :] += jnp.sum(pm * pm, axis=0).reshape(1, -1)
    m = acc[0:1, :] * (1.0 / E)
    v = acc[1:2, :] * (1.0 / E) - m * m
    r = lax.rsqrt(v + EPS)
    cpo = m.shape[1]
    stout_ref[...] = jnp.concatenate(
        [m, r, jnp.zeros((6, cpo), jnp.float32)], axis=0)


def _tc_conv(mode, t1, t2, sta, w2, b2):
    in1, ngh1, st1 = t1
    cpi = in1.shape[1]
    cpo = w2.shape[1]
    in_specs = [
        pl.BlockSpec((BE, cpi), lambda i: (i, 0)),
        pl.BlockSpec((4, BE, cpi), lambda i: (0, i, 0)),
        pl.BlockSpec((8, cpi), lambda i: (0, 0)),
    ]
    args = [in1, ngh1, st1]
    if t2 is not None:
        in_specs = in_specs + [
            pl.BlockSpec((BE, cpi), lambda i: (i, 0)),
            pl.BlockSpec((4, BE, cpi), lambda i: (0, i, 0)),
            pl.BlockSpec((8, cpi), lambda i: (0, 0)),
        ]
        args += list(t2)
    if sta is not None:
        in_specs.append(pl.BlockSpec((8, cpi), lambda i: (0, 0)))
        args.append(sta)
    in_specs += [
        pl.BlockSpec((5 * cpi, cpo), lambda i: (0, 0)),
        pl.BlockSpec((8, cpo), lambda i: (0, 0)),
    ]
    args += [w2, b2]
    return pl.pallas_call(
        lambda *refs: _conv_body(cpi, mode, refs),
        grid=(NBLK,),
        in_specs=in_specs,
        out_specs=[
            pl.BlockSpec((BE, cpo), lambda i: (i, 0)),
            pl.BlockSpec((8, cpo), lambda i: (0, 0)),
        ],
        out_shape=[
            jax.ShapeDtypeStruct((EPAD, cpo), jnp.float32),
            jax.ShapeDtypeStruct((8, cpo), jnp.float32),
        ],
        scratch_shapes=[pltpu.VMEM((8, cpo), jnp.float32)],
    )(*args)


def _stats_body(p1, st1, p2, st2, stout_ref, acc):
    y = _T(p1[...], st1, True) + _T(p2[...], st2, True)
    i = pl.program_id(0)
    rows = i * BE + lax.broadcasted_iota(jnp.int32, (BE, 1), 0)
    ym = jnp.where(rows < E, y, 0.0)

    @pl.when(i == 0)
    def _():
        acc[...] = jnp.zeros_like(acc)

    acc[0:1, :] += jnp.sum(ym, axis=0).reshape(1, -1)
    acc[1:2, :] += jnp.sum(ym * ym, axis=0).reshape(1, -1)
    m = acc[0:1, :] * (1.0 / E)
    v = acc[1:2, :] * (1.0 / E) - m * m
    r = lax.rsqrt(v + EPS)
    cpo = m.shape[1]
    stout_ref[...] = jnp.concatenate(
        [m, r, jnp.zeros((6, cpo), jnp.float32)], axis=0)


def _tc_stats2(p1, st1, p2, st2):
    cp = p1.shape[1]
    return pl.pallas_call(
        _stats_body,
        grid=(NBLK,),
        in_specs=[
            pl.BlockSpec((BE, cp), lambda i: (i, 0)),
            pl.BlockSpec((8, cp), lambda i: (0, 0)),
            pl.BlockSpec((BE, cp), lambda i: (i, 0)),
            pl.BlockSpec((8, cp), lambda i: (0, 0)),
        ],
        out_specs=pl.BlockSpec((8, cp), lambda i: (0, 0)),
        out_shape=jax.ShapeDtypeStruct((8, cp), jnp.float32),
        scratch_shapes=[pltpu.VMEM((8, cp), jnp.float32)],
    )(p1, st1, p2, st2)


def _buildv_body(fe, nvsp, out, win_v, nvs_v, out_v, sem):
    w = lax.axis_index("s") * 2 + lax.axis_index("c")
    v0 = w * VW
    starts = []
    for k in range(6):
        st = ((v0 + k * V) // 2) & ~7
        starts.append(st)
        pltpu.async_copy(fe.at[pl.ds(st, WIN)], win_v.at[k], sem)
    pltpu.sync_copy(nvsp.at[pl.ds(v0, VW)], nvs_v)
    for k in range(6):
        pltpu.make_async_copy(fe.at[pl.ds(starts[k], WIN)], win_v.at[k],
                              sem).wait()

    def it(t, carry):
        vv = v0 + t * 16 + lax.iota(jnp.int32, 16)
        invn = nvs_v[pl.ds(t * 16, 16)]
        for c in range(3):
            acc = jnp.zeros((16,), jnp.float32)
            for k in range(6):
                j = vv + k * V
                valid = j < 2 * E
                e_loc = jnp.where(valid, (j >> 1) - starts[k], 0)
                ch = (j & 1) * 3 + c
                kv = jnp.full((16,), k, jnp.int32)
                vals = plsc.load_gather(win_v, [kv, e_loc, ch])
                acc = acc + jnp.where(valid, vals, 0.0)
            acc = acc / invn
            oidx = (t * 16 + lax.iota(jnp.int32, 16)) * 8 + c
            plsc.store_scatter(out_v, [oidx], acc)
        return carry

    lax.fori_loop(0, VW // 16, it, 0)
    pltpu.sync_copy(out_v, out.at[w])


def _sc_buildv(fe, nvsp):
    k = pl.kernel(
        _buildv_body,
        out_type=jax.ShapeDtypeStruct((NW, VW * 8), jnp.float32),
        mesh=plsc.VectorSubcoreMesh(**_SC_MESH),
        scratch_types=[
            pltpu.VMEM((6, WIN, 16), jnp.float32),
            pltpu.VMEM((VW,), jnp.float32),
            pltpu.VMEM((VW * 8,), jnp.float32),
            pltpu.SemaphoreType.DMA,
        ],
        compiler_params=pltpu.CompilerParams(use_tc_tiling_on_sc=False,
                                             needs_layout_passes=False),
    )
    return k(fe, nvsp)


def _prep_w(w, b):
    cout, cin, _ = w.shape
    cpi, cpo = _pad_c(cin), _pad_c(cout)
    w2 = jnp.zeros((5, cpi, cpo), jnp.float32)
    w2 = w2.at[:, :cin, :cout].set(jnp.transpose(w, (2, 1, 0)))
    b2 = jnp.zeros((8, cpo), jnp.float32).at[0, :cout].set(b)
    return w2.reshape(5 * cpi, cpo), b2


def kernel(x, t, gemm, vei, ve_in, nvsi, nvsin, nvs, params):
    del t, vei, ve_in, nvsi, nvsin
    idx3 = (jnp.zeros((4, EPAD), jnp.int32).at[:, :E].set(gemm[:, 1:].T)
            .reshape(NW, NCHUNK, CHUNK))
    act0 = jnp.zeros((EPAD, 16), jnp.float32).at[:E, :6].set(x[0].T)
    st_id = jnp.zeros((8, 16), jnp.float32).at[1, :].set(1.0)

    def g(table):
        return _sc_gather([table], idx3)[0]

    def conv(mode, t1, t2, sta, p):
        w2, b2 = _prep_w(p['w'], p['b'])
        return _tc_conv(mode, t1, t2, sta, w2, b2)

    blocks = list(params['down']) + list(params['up']) + [params['final']]

    p0 = blocks[0]
    ngh_a = g(act0)
    p1, s1 = conv('raw', (act0, ngh_a, st_id), None, None, p0['c1'])
    ngh1 = g(p1)
    p2, s2 = conv('norm', (p1, ngh1, s1), None, None, p0['blocks'][0]['conv'])

    for p in blocks[1:]:
        ngh2 = g(p2)
        p1n, s1n = conv('norm2', (p1, ngh1, s1), (p2, ngh2, s2), None,
                        p['c1'])
        ngh1n = g(p1n)
        p2n, s2n = conv('norm', (p1n, ngh1n, s1n), None, None,
                        p['blocks'][0]['conv'])
        p1, s1, p2, s2, ngh1 = p1n, s1n, p2n, s2n, ngh1n

    sta = _tc_stats2(p1, s1, p2, s2)
    ngh2 = g(p2)
    fe, _ = conv('norm2_inorm', (p1, ngh1, s1), (p2, ngh2, s2), sta,
                 params['last'])

    nvsp = jnp.ones((VPAD,), jnp.float32).at[:V].set(nvs)
    outb = _sc_buildv(fe, nvsp)
    return outb.reshape(VPAD, 8)[:V, :3][None].astype(jnp.float32)


# speedup vs baseline: 1.3548x; 1.1780x over previous
import functools

import jax
import jax.numpy as jnp
from jax import lax
from jax.experimental import pallas as pl
from jax.experimental.pallas import tpu as pltpu
from jax.experimental.pallas import tpu_sc as plsc

E = 50000
V = 16667
EPAD = 50176
BE = 1024
NBLK = EPAD // BE
NW = 32
RPW = 4 * EPAD // NW
CHUNK = 128
NCHUNK = RPW // CHUNK
VW = 528
VPAD = NW * VW
WIN = 272
EPS = 1e-5

_SC_MESH = dict(core_axis_name="c", subcore_axis_name="s",
                num_cores=2, num_subcores=16)


def _pad_c(c):
    return max(16, c)


def _gather_body(ntab, *refs):
    assert ntab == 1
    (tab, idxs, out, idx_v, rows0, rows1, sem0, sem1) = refs
    w = lax.axis_index("s") * 2 + lax.axis_index("c")
    pltpu.sync_copy(idxs.at[w], idx_v)
    last = NCHUNK - 1

    def start(ci, rows, sem):
        ci = jnp.minimum(ci, last)
        return pltpu.async_copy(tab.at[idx_v.at[ci]], rows, sem)

    def store(ci, rows):
        pltpu.sync_copy(rows, out.at[pl.ds(w * RPW + ci * CHUNK, CHUNK)])

    start(jnp.int32(0), rows0, sem0)
    start(jnp.int32(1), rows1, sem1)

    def step(gi, carry):
        a = 2 * gi
        pltpu.make_async_copy(tab.at[idx_v.at[a]], rows0, sem0).wait()
        store(a, rows0)
        start(a + 2, rows0, sem0)
        b = a + 1
        pltpu.make_async_copy(tab.at[idx_v.at[b]], rows1, sem1).wait()
        store(b, rows1)
        start(a + 3, rows1, sem1)
        return carry

    lax.fori_loop(0, (NCHUNK - 1) // 2, step, 0)
    pltpu.make_async_copy(tab.at[idx_v.at[last]], rows0, sem0).wait()
    store(last, rows0)
    pltpu.make_async_copy(tab.at[idx_v.at[last]], rows1, sem1).wait()


def _sc_gather(tables, idx3):
    assert len(tables) == 1
    cp = tables[0].shape[1]
    k = pl.kernel(
        functools.partial(_gather_body, 1),
        out_type=jax.ShapeDtypeStruct((4 * EPAD, cp), jnp.float32),
        mesh=plsc.VectorSubcoreMesh(**_SC_MESH),
        scratch_types=[
            pltpu.VMEM((NCHUNK, CHUNK), jnp.int32),
            pltpu.VMEM((CHUNK, cp), jnp.float32),
            pltpu.VMEM((CHUNK, cp), jnp.float32),
            pltpu.SemaphoreType.DMA,
            pltpu.SemaphoreType.DMA,
        ],
        compiler_params=pltpu.CompilerParams(use_tc_tiling_on_sc=False),
    )
    return [k(tables[0], idx3).reshape(4, EPAD, cp)]


def _T(g, st_ref, relu):
    y = (g - st_ref[0:1, :]) * st_ref[1:2, :]
    return jnp.maximum(y, 0.0) if relu else y


def _conv_body(cpi, mode, refs):
    if mode in ('norm2', 'norm2_inorm'):
        (in1, ngh1, st1, in2, ngh2, st2, *rest) = refs
    else:
        (in1, ngh1, st1, *rest) = refs
        in2 = ngh2 = st2 = None
    if mode == 'norm2_inorm':
        sta, w2_ref, b_ref, pre_ref, stout_ref, acc = rest
    else:
        sta = None
        w2_ref, b_ref, pre_ref, stout_ref, acc = rest

    def feat(k):
        if k == 0:
            g1 = in1[...]
            g2 = in2[...] if in2 is not None else None
        else:
            g1 = ngh1[k - 1]
            g2 = ngh2[k - 1] if ngh2 is not None else None
        if mode == 'raw':
            return g1
        if mode == 'norm':
            return _T(g1, st1, True)
        a = _T(g1, st1, True) + _T(g2, st2, True)
        if mode == 'norm2_inorm':
            a = (a - sta[0:1, :]) * sta[1:2, :]
        return a

    f0, f1, f2, f3, f4 = [feat(k) for k in range(5)]
    w2 = w2_ref[...]

    def dot(a, wk):
        return lax.dot_general(a, wk, (((1,), (0,)), ((), ())),
                               preferred_element_type=jnp.float32)

    pre = (dot(f0, w2[0:cpi])
           + dot(f1 + f3, w2[cpi:2 * cpi])
           + dot(f2 + f4, w2[2 * cpi:3 * cpi])
           + dot(jnp.abs(f1 - f3), w2[3 * cpi:4 * cpi])
           + dot(jnp.abs(f2 - f4), w2[4 * cpi:5 * cpi]))
    pre = pre + b_ref[0:1, :]
    pre_ref[...] = pre
    i = pl.program_id(0)
    rows = i * BE + lax.broadcasted_iota(jnp.int32, (BE, 1), 0)
    pm = jnp.where(rows < E, pre, 0.0)

    @pl.when(i == 0)
    def _():
        acc[...] = jnp.zeros_like(acc)

    acc[0:1, :] += jnp.sum(pm, axis=0).reshape(1, -1)
    acc[1:2, :] += jnp.sum(pm * pm, axis=0).reshape(1, -1)
    m = acc[0:1, :] * (1.0 / E)
    v = acc[1:2, :] * (1.0 / E) - m * m
    r = lax.rsqrt(v + EPS)
    cpo = m.shape[1]
    stout_ref[...] = jnp.concatenate(
        [m, r, jnp.zeros((6, cpo), jnp.float32)], axis=0)


def _tc_conv(mode, t1, t2, sta, w2, b2):
    in1, ngh1, st1 = t1
    cpi = in1.shape[1]
    cpo = w2.shape[1]
    in_specs = [
        pl.BlockSpec((BE, cpi), lambda i: (i, 0)),
        pl.BlockSpec((4, BE, cpi), lambda i: (0, i, 0)),
        pl.BlockSpec((8, cpi), lambda i: (0, 0)),
    ]
    args = [in1, ngh1, st1]
    if t2 is not None:
        in_specs = in_specs + [
            pl.BlockSpec((BE, cpi), lambda i: (i, 0)),
            pl.BlockSpec((4, BE, cpi), lambda i: (0, i, 0)),
            pl.BlockSpec((8, cpi), lambda i: (0, 0)),
        ]
        args += list(t2)
    if sta is not None:
        in_specs.append(pl.BlockSpec((8, cpi), lambda i: (0, 0)))
        args.append(sta)
    in_specs += [
        pl.BlockSpec((5 * cpi, cpo), lambda i: (0, 0)),
        pl.BlockSpec((8, cpo), lambda i: (0, 0)),
    ]
    args += [w2, b2]
    return pl.pallas_call(
        lambda *refs: _conv_body(cpi, mode, refs),
        grid=(NBLK,),
        in_specs=in_specs,
        out_specs=[
            pl.BlockSpec((BE, cpo), lambda i: (i, 0)),
            pl.BlockSpec((8, cpo), lambda i: (0, 0)),
        ],
        out_shape=[
            jax.ShapeDtypeStruct((EPAD, cpo), jnp.float32),
            jax.ShapeDtypeStruct((8, cpo), jnp.float32),
        ],
        scratch_shapes=[pltpu.VMEM((8, cpo), jnp.float32)],
    )(*args)


def _stats_body(p1, st1, p2, st2, stout_ref, acc):
    y = _T(p1[...], st1, True) + _T(p2[...], st2, True)
    i = pl.program_id(0)
    rows = i * BE + lax.broadcasted_iota(jnp.int32, (BE, 1), 0)
    ym = jnp.where(rows < E, y, 0.0)

    @pl.when(i == 0)
    def _():
        acc[...] = jnp.zeros_like(acc)

    acc[0:1, :] += jnp.sum(ym, axis=0).reshape(1, -1)
    acc[1:2, :] += jnp.sum(ym * ym, axis=0).reshape(1, -1)
    m = acc[0:1, :] * (1.0 / E)
    v = acc[1:2, :] * (1.0 / E) - m * m
    r = lax.rsqrt(v + EPS)
    cpo = m.shape[1]
    stout_ref[...] = jnp.concatenate(
        [m, r, jnp.zeros((6, cpo), jnp.float32)], axis=0)


def _tc_stats2(p1, st1, p2, st2):
    cp = p1.shape[1]
    return pl.pallas_call(
        _stats_body,
        grid=(NBLK,),
        in_specs=[
            pl.BlockSpec((BE, cp), lambda i: (i, 0)),
            pl.BlockSpec((8, cp), lambda i: (0, 0)),
            pl.BlockSpec((BE, cp), lambda i: (i, 0)),
            pl.BlockSpec((8, cp), lambda i: (0, 0)),
        ],
        out_specs=pl.BlockSpec((8, cp), lambda i: (0, 0)),
        out_shape=jax.ShapeDtypeStruct((8, cp), jnp.float32),
        scratch_shapes=[pltpu.VMEM((8, cp), jnp.float32)],
    )(p1, st1, p2, st2)


def _buildv_body(fe, nvsp, out, win_v, nvs_v, out_v, sem):
    w = lax.axis_index("s") * 2 + lax.axis_index("c")
    v0 = w * VW
    starts = []
    for k in range(6):
        st = ((v0 + k * V) // 2) & ~7
        starts.append(st)
        pltpu.async_copy(fe.at[pl.ds(st, WIN)], win_v.at[k], sem)
    pltpu.sync_copy(nvsp.at[pl.ds(v0, VW)], nvs_v)
    for k in range(6):
        pltpu.make_async_copy(fe.at[pl.ds(starts[k], WIN)], win_v.at[k],
                              sem).wait()

    def it(t, carry):
        vv = v0 + t * 16 + lax.iota(jnp.int32, 16)
        invn = nvs_v[pl.ds(t * 16, 16)]
        for c in range(3):
            acc = jnp.zeros((16,), jnp.float32)
            for k in range(6):
                j = vv + k * V
                valid = j < 2 * E
                e_loc = jnp.where(valid, (j >> 1) - starts[k], 0)
                ch = (j & 1) * 3 + c
                kv = jnp.full((16,), k, jnp.int32)
                vals = plsc.load_gather(win_v, [kv, e_loc, ch])
                acc = acc + jnp.where(valid, vals, 0.0)
            acc = acc / invn
            oidx = (t * 16 + lax.iota(jnp.int32, 16)) * 8 + c
            plsc.store_scatter(out_v, [oidx], acc)
        return carry

    lax.fori_loop(0, VW // 16, it, 0)
    pltpu.sync_copy(out_v, out.at[w])


def _sc_buildv(fe, nvsp):
    k = pl.kernel(
        _buildv_body,
        out_type=jax.ShapeDtypeStruct((NW, VW * 8), jnp.float32),
        mesh=plsc.VectorSubcoreMesh(**_SC_MESH),
        scratch_types=[
            pltpu.VMEM((6, WIN, 16), jnp.float32),
            pltpu.VMEM((VW,), jnp.float32),
            pltpu.VMEM((VW * 8,), jnp.float32),
            pltpu.SemaphoreType.DMA,
        ],
        compiler_params=pltpu.CompilerParams(use_tc_tiling_on_sc=False,
                                             needs_layout_passes=False),
    )
    return k(fe, nvsp)


def _prep_w(w, b):
    cout, cin, _ = w.shape
    cpi, cpo = _pad_c(cin), _pad_c(cout)
    w2 = jnp.zeros((5, cpi, cpo), jnp.float32)
    w2 = w2.at[:, :cin, :cout].set(jnp.transpose(w, (2, 1, 0)))
    b2 = jnp.zeros((8, cpo), jnp.float32).at[0, :cout].set(b)
    return w2.reshape(5 * cpi, cpo), b2


def kernel(x, t, gemm, vei, ve_in, nvsi, nvsin, nvs, params):
    del t, vei, ve_in, nvsi, nvsin
    idx3 = (jnp.zeros((4, EPAD), jnp.int32).at[:, :E].set(gemm[:, 1:].T)
            .reshape(NW, NCHUNK, CHUNK))
    act0 = jnp.zeros((EPAD, 16), jnp.float32).at[:E, :6].set(x[0].T)
    st_id = jnp.zeros((8, 16), jnp.float32).at[1, :].set(1.0)

    def g(table):
        return _sc_gather([table], idx3)[0]

    def conv(mode, t1, t2, sta, p):
        w2, b2 = _prep_w(p['w'], p['b'])
        return _tc_conv(mode, t1, t2, sta, w2, b2)

    blocks = list(params['down']) + list(params['up']) + [params['final']]

    p0 = blocks[0]
    ngh_a = g(act0)
    p1, s1 = conv('raw', (act0, ngh_a, st_id), None, None, p0['c1'])
    ngh1 = g(p1)
    p2, s2 = conv('norm', (p1, ngh1, s1), None, None, p0['blocks'][0]['conv'])

    for p in blocks[1:]:
        ngh2 = g(p2)
        p1n, s1n = conv('norm2', (p1, ngh1, s1), (p2, ngh2, s2), None,
                        p['c1'])
        ngh1n = g(p1n)
        p2n, s2n = conv('norm', (p1n, ngh1n, s1n), None, None,
                        p['blocks'][0]['conv'])
        p1, s1, p2, s2, ngh1 = p1n, s1n, p2n, s2n, ngh1n

    sta = _tc_stats2(p1, s1, p2, s2)
    ngh2 = g(p2)
    fe, _ = conv('norm2_inorm', (p1, ngh1, s1), (p2, ngh2, s2), sta,
                 params['last'])

    nvsp = jnp.ones((VPAD,), jnp.float32).at[:V].set(nvs)
    outb = _sc_buildv(fe, nvsp)
    return outb.reshape(VPAD, 8)[:V, :3][None].astype(jnp.float32)


# speedup vs baseline: 1.4262x; 1.0527x over previous
import functools

import jax
import jax.numpy as jnp
from jax import lax
from jax.experimental import pallas as pl
from jax.experimental.pallas import tpu as pltpu
from jax.experimental.pallas import tpu_sc as plsc

E = 50000
V = 16667
EPAD = 50176
BE = 1792
NBLK = EPAD // BE
NW = 32
RPW = 4 * EPAD // NW
CHUNK = 128
NCHUNK = RPW // CHUNK
VW = 528
VPAD = NW * VW
WIN = 272
EPS = 1e-5

_SC_MESH = dict(core_axis_name="c", subcore_axis_name="s",
                num_cores=2, num_subcores=16)


def _pad_c(c):
    return max(16, c)


def _gather_body(ntab, *refs):
    assert ntab == 1
    (tab, idxs, out, idx_v, rows0, rows1, sem0, sem1) = refs
    w = lax.axis_index("s") * 2 + lax.axis_index("c")
    pltpu.sync_copy(idxs.at[w], idx_v)
    last = NCHUNK - 1

    def start(ci, rows, sem):
        ci = jnp.minimum(ci, last)
        return pltpu.async_copy(tab.at[idx_v.at[ci]], rows, sem)

    def store(ci, rows):
        pltpu.sync_copy(rows, out.at[pl.ds(w * RPW + ci * CHUNK, CHUNK)])

    start(jnp.int32(0), rows0, sem0)
    start(jnp.int32(1), rows1, sem1)

    def step(gi, carry):
        a = 2 * gi
        pltpu.make_async_copy(tab.at[idx_v.at[a]], rows0, sem0).wait()
        store(a, rows0)
        start(a + 2, rows0, sem0)
        b = a + 1
        pltpu.make_async_copy(tab.at[idx_v.at[b]], rows1, sem1).wait()
        store(b, rows1)
        start(a + 3, rows1, sem1)
        return carry

    lax.fori_loop(0, (NCHUNK - 1) // 2, step, 0)
    pltpu.make_async_copy(tab.at[idx_v.at[last]], rows0, sem0).wait()
    store(last, rows0)
    pltpu.make_async_copy(tab.at[idx_v.at[last]], rows1, sem1).wait()


def _sc_gather(tables, idx3):
    assert len(tables) == 1
    cp = tables[0].shape[1]
    k = pl.kernel(
        functools.partial(_gather_body, 1),
        out_type=jax.ShapeDtypeStruct((4 * EPAD, cp), jnp.float32),
        mesh=plsc.VectorSubcoreMesh(**_SC_MESH),
        scratch_types=[
            pltpu.VMEM((NCHUNK, CHUNK), jnp.int32),
            pltpu.VMEM((CHUNK, cp), jnp.float32),
            pltpu.VMEM((CHUNK, cp), jnp.float32),
            pltpu.SemaphoreType.DMA,
            pltpu.SemaphoreType.DMA,
        ],
        compiler_params=pltpu.CompilerParams(use_tc_tiling_on_sc=False),
    )
    return [k(tables[0], idx3).reshape(4, EPAD, cp)]


def _T(g, st_ref, relu):
    y = (g - st_ref[0:1, :]) * st_ref[1:2, :]
    return jnp.maximum(y, 0.0) if relu else y


def _conv_body(cpi, mode, refs):
    if mode in ('norm2', 'norm2_inorm'):
        (in1, ngh1, st1, in2, ngh2, st2, *rest) = refs
    else:
        (in1, ngh1, st1, *rest) = refs
        in2 = ngh2 = st2 = None
    if mode == 'norm2_inorm':
        sta, w2_ref, b_ref, pre_ref, stout_ref, acc = rest
    else:
        sta = None
        w2_ref, b_ref, pre_ref, stout_ref, acc = rest

    def feat(k):
        if k == 0:
            g1 = in1[...]
            g2 = in2[...] if in2 is not None else None
        else:
            g1 = ngh1[k - 1]
            g2 = ngh2[k - 1] if ngh2 is not None else None
        if mode == 'raw':
            return g1
        if mode == 'norm':
            return _T(g1, st1, True)
        a = _T(g1, st1, True) + _T(g2, st2, True)
        if mode == 'norm2_inorm':
            a = (a - sta[0:1, :]) * sta[1:2, :]
        return a

    f0, f1, f2, f3, f4 = [feat(k) for k in range(5)]
    w2 = w2_ref[...]

    def dot(a, wk):
        return lax.dot_general(a, wk, (((1,), (0,)), ((), ())),
                               preferred_element_type=jnp.float32)

    pre = (dot(f0, w2[0:cpi])
           + dot(f1 + f3, w2[cpi:2 * cpi])
           + dot(f2 + f4, w2[2 * cpi:3 * cpi])
           + dot(jnp.abs(f1 - f3), w2[3 * cpi:4 * cpi])
           + dot(jnp.abs(f2 - f4), w2[4 * cpi:5 * cpi]))
    pre = pre + b_ref[0:1, :]
    pre_ref[...] = pre
    i = pl.program_id(0)
    rows = i * BE + lax.broadcasted_iota(jnp.int32, (BE, 1), 0)
    pm = jnp.where(rows < E, pre, 0.0)

    @pl.when(i == 0)
    def _():
        acc[...] = jnp.zeros_like(acc)

    acc[0:1, :] += jnp.sum(pm, axis=0).reshape(1, -1)
    acc[1:2, :] += jnp.sum(pm * pm, axis=0).reshape(1, -1)
    m = acc[0:1, :] * (1.0 / E)
    v = acc[1:2, :] * (1.0 / E) - m * m
    r = lax.rsqrt(v + EPS)
    cpo = m.shape[1]
    stout_ref[...] = jnp.concatenate(
        [m, r, jnp.zeros((6, cpo), jnp.float32)], axis=0)


def _tc_conv(mode, t1, t2, sta, w2, b2):
    in1, ngh1, st1 = t1
    cpi = in1.shape[1]
    cpo = w2.shape[1]
    in_specs = [
        pl.BlockSpec((BE, cpi), lambda i: (i, 0)),
        pl.BlockSpec((4, BE, cpi), lambda i: (0, i, 0)),
        pl.BlockSpec((8, cpi), lambda i: (0, 0)),
    ]
    args = [in1, ngh1, st1]
    if t2 is not None:
        in_specs = in_specs + [
            pl.BlockSpec((BE, cpi), lambda i: (i, 0)),
            pl.BlockSpec((4, BE, cpi), lambda i: (0, i, 0)),
            pl.BlockSpec((8, cpi), lambda i: (0, 0)),
        ]
        args += list(t2)
    if sta is not None:
        in_specs.append(pl.BlockSpec((8, cpi), lambda i: (0, 0)))
        args.append(sta)
    in_specs += [
        pl.BlockSpec((5 * cpi, cpo), lambda i: (0, 0)),
        pl.BlockSpec((8, cpo), lambda i: (0, 0)),
    ]
    args += [w2, b2]
    return pl.pallas_call(
        lambda *refs: _conv_body(cpi, mode, refs),
        grid=(NBLK,),
        in_specs=in_specs,
        out_specs=[
            pl.BlockSpec((BE, cpo), lambda i: (i, 0)),
            pl.BlockSpec((8, cpo), lambda i: (0, 0)),
        ],
        out_shape=[
            jax.ShapeDtypeStruct((EPAD, cpo), jnp.float32),
            jax.ShapeDtypeStruct((8, cpo), jnp.float32),
        ],
        scratch_shapes=[pltpu.VMEM((8, cpo), jnp.float32)],
    )(*args)


def _stats_body(p1, st1, p2, st2, stout_ref, acc):
    y = _T(p1[...], st1, True) + _T(p2[...], st2, True)
    i = pl.program_id(0)
    rows = i * BE + lax.broadcasted_iota(jnp.int32, (BE, 1), 0)
    ym = jnp.where(rows < E, y, 0.0)

    @pl.when(i == 0)
    def _():
        acc[...] = jnp.zeros_like(acc)

    acc[0:1, :] += jnp.sum(ym, axis=0).reshape(1, -1)
    acc[1:2, :] += jnp.sum(ym * ym, axis=0).reshape(1, -1)
    m = acc[0:1, :] * (1.0 / E)
    v = acc[1:2, :] * (1.0 / E) - m * m
    r = lax.rsqrt(v + EPS)
    cpo = m.shape[1]
    stout_ref[...] = jnp.concatenate(
        [m, r, jnp.zeros((6, cpo), jnp.float32)], axis=0)


def _tc_stats2(p1, st1, p2, st2):
    cp = p1.shape[1]
    return pl.pallas_call(
        _stats_body,
        grid=(NBLK,),
        in_specs=[
            pl.BlockSpec((BE, cp), lambda i: (i, 0)),
            pl.BlockSpec((8, cp), lambda i: (0, 0)),
            pl.BlockSpec((BE, cp), lambda i: (i, 0)),
            pl.BlockSpec((8, cp), lambda i: (0, 0)),
        ],
        out_specs=pl.BlockSpec((8, cp), lambda i: (0, 0)),
        out_shape=jax.ShapeDtypeStruct((8, cp), jnp.float32),
        scratch_shapes=[pltpu.VMEM((8, cp), jnp.float32)],
    )(p1, st1, p2, st2)


def _buildv_body(fe, nvsp, out, win_v, nvs_v, out_v, sem):
    w = lax.axis_index("s") * 2 + lax.axis_index("c")
    v0 = w * VW
    starts = []
    for k in range(6):
        st = ((v0 + k * V) // 2) & ~7
        starts.append(st)
        pltpu.async_copy(fe.at[pl.ds(st, WIN)], win_v.at[k], sem)
    pltpu.sync_copy(nvsp.at[pl.ds(v0, VW)], nvs_v)
    for k in range(6):
        pltpu.make_async_copy(fe.at[pl.ds(starts[k], WIN)], win_v.at[k],
                              sem).wait()

    def it(t, carry):
        vv = v0 + t * 16 + lax.iota(jnp.int32, 16)
        invn = nvs_v[pl.ds(t * 16, 16)]
        for c in range(3):
            acc = jnp.zeros((16,), jnp.float32)
            for k in range(6):
                j = vv + k * V
                valid = j < 2 * E
                e_loc = jnp.where(valid, (j >> 1) - starts[k], 0)
                ch = (j & 1) * 3 + c
                kv = jnp.full((16,), k, jnp.int32)
                vals = plsc.load_gather(win_v, [kv, e_loc, ch])
                acc = acc + jnp.where(valid, vals, 0.0)
            acc = acc / invn
            oidx = (t * 16 + lax.iota(jnp.int32, 16)) * 8 + c
            plsc.store_scatter(out_v, [oidx], acc)
        return carry

    lax.fori_loop(0, VW // 16, it, 0)
    pltpu.sync_copy(out_v, out.at[w])


def _sc_buildv(fe, nvsp):
    k = pl.kernel(
        _buildv_body,
        out_type=jax.ShapeDtypeStruct((NW, VW * 8), jnp.float32),
        mesh=plsc.VectorSubcoreMesh(**_SC_MESH),
        scratch_types=[
            pltpu.VMEM((6, WIN, 16), jnp.float32),
            pltpu.VMEM((VW,), jnp.float32),
            pltpu.VMEM((VW * 8,), jnp.float32),
            pltpu.SemaphoreType.DMA,
        ],
        compiler_params=pltpu.CompilerParams(use_tc_tiling_on_sc=False,
                                             needs_layout_passes=False),
    )
    return k(fe, nvsp)


def _prep_w(w, b):
    cout, cin, _ = w.shape
    cpi, cpo = _pad_c(cin), _pad_c(cout)
    w2 = jnp.zeros((5, cpi, cpo), jnp.float32)
    w2 = w2.at[:, :cin, :cout].set(jnp.transpose(w, (2, 1, 0)))
    b2 = jnp.zeros((8, cpo), jnp.float32).at[0, :cout].set(b)
    return w2.reshape(5 * cpi, cpo), b2


def kernel(x, t, gemm, vei, ve_in, nvsi, nvsin, nvs, params):
    del t, vei, ve_in, nvsi, nvsin
    idx3 = (jnp.zeros((4, EPAD), jnp.int32).at[:, :E].set(gemm[:, 1:].T)
            .reshape(NW, NCHUNK, CHUNK))
    act0 = jnp.zeros((EPAD, 16), jnp.float32).at[:E, :6].set(x[0].T)
    st_id = jnp.zeros((8, 16), jnp.float32).at[1, :].set(1.0)

    def g(table):
        return _sc_gather([table], idx3)[0]

    def conv(mode, t1, t2, sta, p):
        w2, b2 = _prep_w(p['w'], p['b'])
        return _tc_conv(mode, t1, t2, sta, w2, b2)

    blocks = list(params['down']) + list(params['up']) + [params['final']]

    p0 = blocks[0]
    ngh_a = g(act0)
    p1, s1 = conv('raw', (act0, ngh_a, st_id), None, None, p0['c1'])
    ngh1 = g(p1)
    p2, s2 = conv('norm', (p1, ngh1, s1), None, None, p0['blocks'][0]['conv'])

    for p in blocks[1:]:
        ngh2 = g(p2)
        p1n, s1n = conv('norm2', (p1, ngh1, s1), (p2, ngh2, s2), None,
                        p['c1'])
        ngh1n = g(p1n)
        p2n, s2n = conv('norm', (p1n, ngh1n, s1n), None, None,
                        p['blocks'][0]['conv'])
        p1, s1, p2, s2, ngh1 = p1n, s1n, p2n, s2n, ngh1n

    sta = _tc_stats2(p1, s1, p2, s2)
    ngh2 = g(p2)
    fe, _ = conv('norm2_inorm', (p1, ngh1, s1), (p2, ngh2, s2), sta,
                 params['last'])

    nvsp = jnp.ones((VPAD,), jnp.float32).at[:V].set(nvs)
    outb = _sc_buildv(fe, nvsp)
    return outb.reshape(VPAD, 8)[:V, :3][None].astype(jnp.float32)


# speedup vs baseline: 1.4516x; 1.0178x over previous
import functools

import jax
import jax.numpy as jnp
from jax import lax
from jax.experimental import pallas as pl
from jax.experimental.pallas import tpu as pltpu
from jax.experimental.pallas import tpu_sc as plsc

E = 50000
V = 16667
EPAD = 50176
BE = 3584
NBLK = EPAD // BE
NW = 32
RPW = 4 * EPAD // NW
CHUNK = 128
NCHUNK = RPW // CHUNK
VW = 528
VPAD = NW * VW
WIN = 272
EPS = 1e-5

_SC_MESH = dict(core_axis_name="c", subcore_axis_name="s",
                num_cores=2, num_subcores=16)


def _pad_c(c):
    return max(16, c)


def _gather_body(ntab, *refs):
    assert ntab == 1
    (tab, idxs, out, idx_v, rows0, rows1, sem0, sem1) = refs
    w = lax.axis_index("s") * 2 + lax.axis_index("c")
    pltpu.sync_copy(idxs.at[w], idx_v)
    last = NCHUNK - 1

    def start(ci, rows, sem):
        ci = jnp.minimum(ci, last)
        return pltpu.async_copy(tab.at[idx_v.at[ci]], rows, sem)

    def store(ci, rows):
        pltpu.sync_copy(rows, out.at[pl.ds(w * RPW + ci * CHUNK, CHUNK)])

    start(jnp.int32(0), rows0, sem0)
    start(jnp.int32(1), rows1, sem1)

    def step(gi, carry):
        a = 2 * gi
        pltpu.make_async_copy(tab.at[idx_v.at[a]], rows0, sem0).wait()
        store(a, rows0)
        start(a + 2, rows0, sem0)
        b = a + 1
        pltpu.make_async_copy(tab.at[idx_v.at[b]], rows1, sem1).wait()
        store(b, rows1)
        start(a + 3, rows1, sem1)
        return carry

    lax.fori_loop(0, (NCHUNK - 1) // 2, step, 0)
    pltpu.make_async_copy(tab.at[idx_v.at[last]], rows0, sem0).wait()
    store(last, rows0)
    pltpu.make_async_copy(tab.at[idx_v.at[last]], rows1, sem1).wait()


def _sc_gather(tables, idx3):
    assert len(tables) == 1
    cp = tables[0].shape[1]
    k = pl.kernel(
        functools.partial(_gather_body, 1),
        out_type=jax.ShapeDtypeStruct((4 * EPAD, cp), jnp.float32),
        mesh=plsc.VectorSubcoreMesh(**_SC_MESH),
        scratch_types=[
            pltpu.VMEM((NCHUNK, CHUNK), jnp.int32),
            pltpu.VMEM((CHUNK, cp), jnp.float32),
            pltpu.VMEM((CHUNK, cp), jnp.float32),
            pltpu.SemaphoreType.DMA,
            pltpu.SemaphoreType.DMA,
        ],
        compiler_params=pltpu.CompilerParams(use_tc_tiling_on_sc=False),
    )
    return [k(tables[0], idx3).reshape(4, EPAD, cp)]


def _T(g, st_ref, relu):
    y = (g - st_ref[0:1, :]) * st_ref[1:2, :]
    return jnp.maximum(y, 0.0) if relu else y


def _conv_body(cpi, mode, refs):
    if mode in ('norm2', 'norm2_inorm'):
        (in1, ngh1, st1, in2, ngh2, st2, *rest) = refs
    else:
        (in1, ngh1, st1, *rest) = refs
        in2 = ngh2 = st2 = None
    if mode == 'norm2_inorm':
        sta, w2_ref, b_ref, pre_ref, stout_ref, acc = rest
    else:
        sta = None
        w2_ref, b_ref, pre_ref, stout_ref, acc = rest

    def feat(k):
        if k == 0:
            g1 = in1[...]
            g2 = in2[...] if in2 is not None else None
        else:
            g1 = ngh1[k - 1]
            g2 = ngh2[k - 1] if ngh2 is not None else None
        if mode == 'raw':
            return g1
        if mode == 'norm':
            return _T(g1, st1, True)
        a = _T(g1, st1, True) + _T(g2, st2, True)
        if mode == 'norm2_inorm':
            a = (a - sta[0:1, :]) * sta[1:2, :]
        return a

    f0, f1, f2, f3, f4 = [feat(k) for k in range(5)]
    w2 = w2_ref[...]

    def dot(a, wk):
        return lax.dot_general(a, wk, (((1,), (0,)), ((), ())),
                               preferred_element_type=jnp.float32)

    pre = (dot(f0, w2[0:cpi])
           + dot(f1 + f3, w2[cpi:2 * cpi])
           + dot(f2 + f4, w2[2 * cpi:3 * cpi])
           + dot(jnp.abs(f1 - f3), w2[3 * cpi:4 * cpi])
           + dot(jnp.abs(f2 - f4), w2[4 * cpi:5 * cpi]))
    pre = pre + b_ref[0:1, :]
    pre_ref[...] = pre
    i = pl.program_id(0)
    rows = i * BE + lax.broadcasted_iota(jnp.int32, (BE, 1), 0)
    pm = jnp.where(rows < E, pre, 0.0)

    @pl.when(i == 0)
    def _():
        acc[...] = jnp.zeros_like(acc)

    acc[0:1, :] += jnp.sum(pm, axis=0).reshape(1, -1)
    acc[1:2, :] += jnp.sum(pm * pm, axis=0).reshape(1, -1)
    m = acc[0:1, :] * (1.0 / E)
    v = acc[1:2, :] * (1.0 / E) - m * m
    r = lax.rsqrt(v + EPS)
    cpo = m.shape[1]
    stout_ref[...] = jnp.concatenate(
        [m, r, jnp.zeros((6, cpo), jnp.float32)], axis=0)


def _tc_conv(mode, t1, t2, sta, w2, b2):
    in1, ngh1, st1 = t1
    cpi = in1.shape[1]
    cpo = w2.shape[1]
    in_specs = [
        pl.BlockSpec((BE, cpi), lambda i: (i, 0)),
        pl.BlockSpec((4, BE, cpi), lambda i: (0, i, 0)),
        pl.BlockSpec((8, cpi), lambda i: (0, 0)),
    ]
    args = [in1, ngh1, st1]
    if t2 is not None:
        in_specs = in_specs + [
            pl.BlockSpec((BE, cpi), lambda i: (i, 0)),
            pl.BlockSpec((4, BE, cpi), lambda i: (0, i, 0)),
            pl.BlockSpec((8, cpi), lambda i: (0, 0)),
        ]
        args += list(t2)
    if sta is not None:
        in_specs.append(pl.BlockSpec((8, cpi), lambda i: (0, 0)))
        args.append(sta)
    in_specs += [
        pl.BlockSpec((5 * cpi, cpo), lambda i: (0, 0)),
        pl.BlockSpec((8, cpo), lambda i: (0, 0)),
    ]
    args += [w2, b2]
    return pl.pallas_call(
        lambda *refs: _conv_body(cpi, mode, refs),
        grid=(NBLK,),
        in_specs=in_specs,
        out_specs=[
            pl.BlockSpec((BE, cpo), lambda i: (i, 0)),
            pl.BlockSpec((8, cpo), lambda i: (0, 0)),
        ],
        out_shape=[
            jax.ShapeDtypeStruct((EPAD, cpo), jnp.float32),
            jax.ShapeDtypeStruct((8, cpo), jnp.float32),
        ],
        scratch_shapes=[pltpu.VMEM((8, cpo), jnp.float32)],
    )(*args)


def _stats_body(p1, st1, p2, st2, stout_ref, acc):
    y = _T(p1[...], st1, True) + _T(p2[...], st2, True)
    i = pl.program_id(0)
    rows = i * BE + lax.broadcasted_iota(jnp.int32, (BE, 1), 0)
    ym = jnp.where(rows < E, y, 0.0)

    @pl.when(i == 0)
    def _():
        acc[...] = jnp.zeros_like(acc)

    acc[0:1, :] += jnp.sum(ym, axis=0).reshape(1, -1)
    acc[1:2, :] += jnp.sum(ym * ym, axis=0).reshape(1, -1)
    m = acc[0:1, :] * (1.0 / E)
    v = acc[1:2, :] * (1.0 / E) - m * m
    r = lax.rsqrt(v + EPS)
    cpo = m.shape[1]
    stout_ref[...] = jnp.concatenate(
        [m, r, jnp.zeros((6, cpo), jnp.float32)], axis=0)


def _tc_stats2(p1, st1, p2, st2):
    cp = p1.shape[1]
    return pl.pallas_call(
        _stats_body,
        grid=(NBLK,),
        in_specs=[
            pl.BlockSpec((BE, cp), lambda i: (i, 0)),
            pl.BlockSpec((8, cp), lambda i: (0, 0)),
            pl.BlockSpec((BE, cp), lambda i: (i, 0)),
            pl.BlockSpec((8, cp), lambda i: (0, 0)),
        ],
        out_specs=pl.BlockSpec((8, cp), lambda i: (0, 0)),
        out_shape=jax.ShapeDtypeStruct((8, cp), jnp.float32),
        scratch_shapes=[pltpu.VMEM((8, cp), jnp.float32)],
    )(p1, st1, p2, st2)


def _buildv_body(fe, nvsp, out, win_v, nvs_v, out_v, sem):
    w = lax.axis_index("s") * 2 + lax.axis_index("c")
    v0 = w * VW
    starts = []
    for k in range(6):
        st = ((v0 + k * V) // 2) & ~7
        starts.append(st)
        pltpu.async_copy(fe.at[pl.ds(st, WIN)], win_v.at[k], sem)
    pltpu.sync_copy(nvsp.at[pl.ds(v0, VW)], nvs_v)
    for k in range(6):
        pltpu.make_async_copy(fe.at[pl.ds(starts[k], WIN)], win_v.at[k],
                              sem).wait()

    def it(t, carry):
        vv = v0 + t * 16 + lax.iota(jnp.int32, 16)
        invn = nvs_v[pl.ds(t * 16, 16)]
        for c in range(3):
            acc = jnp.zeros((16,), jnp.float32)
            for k in range(6):
                j = vv + k * V
                valid = j < 2 * E
                e_loc = jnp.where(valid, (j >> 1) - starts[k], 0)
                ch = (j & 1) * 3 + c
                kv = jnp.full((16,), k, jnp.int32)
                vals = plsc.load_gather(win_v, [kv, e_loc, ch])
                acc = acc + jnp.where(valid, vals, 0.0)
            acc = acc / invn
            oidx = (t * 16 + lax.iota(jnp.int32, 16)) * 8 + c
            plsc.store_scatter(out_v, [oidx], acc)
        return carry

    lax.fori_loop(0, VW // 16, it, 0)
    pltpu.sync_copy(out_v, out.at[w])


def _sc_buildv(fe, nvsp):
    k = pl.kernel(
        _buildv_body,
        out_type=jax.ShapeDtypeStruct((NW, VW * 8), jnp.float32),
        mesh=plsc.VectorSubcoreMesh(**_SC_MESH),
        scratch_types=[
            pltpu.VMEM((6, WIN, 16), jnp.float32),
            pltpu.VMEM((VW,), jnp.float32),
            pltpu.VMEM((VW * 8,), jnp.float32),
            pltpu.SemaphoreType.DMA,
        ],
        compiler_params=pltpu.CompilerParams(use_tc_tiling_on_sc=False,
                                             needs_layout_passes=False),
    )
    return k(fe, nvsp)


def _prep_w(w, b):
    cout, cin, _ = w.shape
    cpi, cpo = _pad_c(cin), _pad_c(cout)
    w2 = jnp.zeros((5, cpi, cpo), jnp.float32)
    w2 = w2.at[:, :cin, :cout].set(jnp.transpose(w, (2, 1, 0)))
    b2 = jnp.zeros((8, cpo), jnp.float32).at[0, :cout].set(b)
    return w2.reshape(5 * cpi, cpo), b2


def kernel(x, t, gemm, vei, ve_in, nvsi, nvsin, nvs, params):
    del t, vei, ve_in, nvsi, nvsin
    idx3 = (jnp.zeros((4, EPAD), jnp.int32).at[:, :E].set(gemm[:, 1:].T)
            .reshape(NW, NCHUNK, CHUNK))
    act0 = jnp.zeros((EPAD, 16), jnp.float32).at[:E, :6].set(x[0].T)
    st_id = jnp.zeros((8, 16), jnp.float32).at[1, :].set(1.0)

    def g(table):
        return _sc_gather([table], idx3)[0]

    def conv(mode, t1, t2, sta, p):
        w2, b2 = _prep_w(p['w'], p['b'])
        return _tc_conv(mode, t1, t2, sta, w2, b2)

    blocks = list(params['down']) + list(params['up']) + [params['final']]

    p0 = blocks[0]
    ngh_a = g(act0)
    p1, s1 = conv('raw', (act0, ngh_a, st_id), None, None, p0['c1'])
    ngh1 = g(p1)
    p2, s2 = conv('norm', (p1, ngh1, s1), None, None, p0['blocks'][0]['conv'])

    for p in blocks[1:]:
        ngh2 = g(p2)
        p1n, s1n = conv('norm2', (p1, ngh1, s1), (p2, ngh2, s2), None,
                        p['c1'])
        ngh1n = g(p1n)
        p2n, s2n = conv('norm', (p1n, ngh1n, s1n), None, None,
                        p['blocks'][0]['conv'])
        p1, s1, p2, s2, ngh1 = p1n, s1n, p2n, s2n, ngh1n

    sta = _tc_stats2(p1, s1, p2, s2)
    ngh2 = g(p2)
    fe, _ = conv('norm2_inorm', (p1, ngh1, s1), (p2, ngh2, s2), sta,
                 params['last'])

    nvsp = jnp.ones((VPAD,), jnp.float32).at[:V].set(nvs)
    outb = _sc_buildv(fe, nvsp)
    return outb.reshape(VPAD, 8)[:V, :3][None].astype(jnp.float32)
